# Initial kernel scaffold; baseline (speedup 1.0000x reference)
import jax
import jax.numpy as jnp
from jax.experimental import pallas as pl


def kernel(x, edge_index, W1, b1, W2, b2, a, fcW, fcb):
    raise NotImplementedError("write your pallas kernel here")


# speedup vs baseline: 22.2399x; 22.2399x over previous
import jax
import jax.numpy as jnp
from jax import lax
from jax.experimental import pallas as pl
from jax.experimental.pallas import tpu as pltpu
from jax.experimental.pallas import tpu_sc as plsc

N_NODES = 10000
N_EDGES = 320000

NC = 2
NS = 16
NW = NC * NS

E_PER_W = N_EDGES // NW
CHUNK = 125
ROWS_PER_W = E_PER_W // CHUNK
SLAB = 624
TAIL_BASE = SLAB * NS
TAIL = N_NODES - TAIL_BASE


def _sc_mesh():
    return plsc.VectorSubcoreMesh(core_axis_name="c", subcore_axis_name="s")


def _deg_body(dst_hbm, hist_hbm, dst_v, hist_v):
    wid = lax.axis_index("s") * NC + lax.axis_index("c")
    ones = jnp.ones((16,), jnp.float32)
    zeros = jnp.zeros((16,), jnp.float32)

    @pl.loop(0, N_NODES // 16)
    def _(i):
        hist_v[pl.ds(i * 16, 16)] = zeros

    pltpu.sync_copy(dst_hbm.at[pl.ds(wid * E_PER_W, E_PER_W)], dst_v)

    @pl.loop(0, E_PER_W // 16)
    def _(i):
        idx = dst_v[pl.ds(i * 16, 16)]
        plsc.addupdate_scatter(hist_v, [idx], ones)

    pltpu.sync_copy(hist_v, hist_hbm.at[wid])


def _sc_degree(dst):
    return pl.kernel(
        _deg_body,
        out_type=jax.ShapeDtypeStruct((NW, N_NODES), jnp.float32),
        mesh=_sc_mesh(),
        scratch_types=[
            pltpu.VMEM((E_PER_W,), jnp.int32),
            pltpu.VMEM((N_NODES,), jnp.float32),
        ],
        compiler_params=pltpu.CompilerParams(needs_layout_passes=False),
    )(dst)


def _agg_body(h_hbm, src_hbm, dst_hbm, out_hbm, acc, src_v, dst_v, rbuf, sem):
    cid = lax.axis_index("c")
    sid = lax.axis_index("s")
    wid = sid * NC + cid

    nbase = sid * SLAB
    pltpu.sync_copy(h_hbm.at[pl.ds(nbase, SLAB)], acc.at[pl.ds(nbase, SLAB)])

    @pl.when(sid == NS - 1)
    def _():
        pltpu.sync_copy(h_hbm.at[pl.ds(TAIL_BASE, TAIL)],
                        acc.at[pl.ds(TAIL_BASE, TAIL)])

    rbase = wid * ROWS_PER_W
    pltpu.sync_copy(src_hbm.at[pl.ds(rbase, ROWS_PER_W)], src_v)
    pltpu.sync_copy(dst_hbm.at[pl.ds(rbase, ROWS_PER_W)], dst_v)

    plsc.subcore_barrier()

    @pl.loop(0, ROWS_PER_W)
    def _(j):
        pltpu.async_copy(h_hbm.at[src_v.at[j]], rbuf, sem).wait()
        pltpu.sync_copy(rbuf, acc.at[dst_v.at[j]], add=True)

    plsc.subcore_barrier()

    pltpu.sync_copy(acc.at[pl.ds(nbase, SLAB)],
                    out_hbm.at[cid, pl.ds(nbase, SLAB)])

    @pl.when(sid == NS - 1)
    def _():
        pltpu.sync_copy(acc.at[pl.ds(TAIL_BASE, TAIL)],
                        out_hbm.at[cid, pl.ds(TAIL_BASE, TAIL)])


def _sc_aggregate(h, src2d, dst2d):
    d = h.shape[1]
    return pl.kernel(
        _agg_body,
        out_type=jax.ShapeDtypeStruct((NC, N_NODES, d), jnp.float32),
        mesh=_sc_mesh(),
        scratch_types=[
            pltpu.VMEM_SHARED((N_NODES, d), jnp.float32),
            pltpu.VMEM((ROWS_PER_W, CHUNK), jnp.int32),
            pltpu.VMEM((ROWS_PER_W, CHUNK), jnp.int32),
            pltpu.VMEM((CHUNK, d), jnp.float32),
            pltpu.SemaphoreType.DMA,
        ],
    )(h, src2d, dst2d)


_ECHUNK = 2000
E_HALF = N_EDGES // 2


def _agg16_body(ht_hbm, src_hbm, dst_hbm, out_hbm, tbl_v, acc_v, src_v, dst_v):
    cid = lax.axis_index("c")
    sid = lax.axis_index("s")
    wid = sid * NC + cid
    col = wid // 2
    half = wid % 2

    zeros = jnp.zeros((16,), jnp.float32)

    @pl.loop(0, N_NODES // 16)
    def _(i):
        acc_v[pl.ds(i * 16, 16)] = zeros

    pltpu.sync_copy(ht_hbm.at[col], tbl_v)

    @pl.loop(0, E_HALF // _ECHUNK)
    def _(k):
        ebase = half * E_HALF + k * _ECHUNK
        pltpu.sync_copy(src_hbm.at[pl.ds(ebase, _ECHUNK)], src_v)
        pltpu.sync_copy(dst_hbm.at[pl.ds(ebase, _ECHUNK)], dst_v)

        @pl.loop(0, _ECHUNK // 16)
        def _(i):
            idx_s = src_v[pl.ds(i * 16, 16)]
            vals = plsc.load_gather(tbl_v, [idx_s])
            idx_d = dst_v[pl.ds(i * 16, 16)]
            plsc.addupdate_scatter(acc_v, [idx_d], vals)

    pltpu.sync_copy(acc_v, out_hbm.at[half, col])


def _sc_aggregate16(ht, src, dst):
    h2 = ht.shape[0]
    return pl.kernel(
        _agg16_body,
        out_type=jax.ShapeDtypeStruct((2, h2, N_NODES), jnp.float32),
        mesh=_sc_mesh(),
        scratch_types=[
            pltpu.VMEM((N_NODES,), jnp.float32),
            pltpu.VMEM((N_NODES,), jnp.float32),
            pltpu.VMEM((_ECHUNK,), jnp.int32),
            pltpu.VMEM((_ECHUNK,), jnp.int32),
        ],
        compiler_params=pltpu.CompilerParams(needs_layout_passes=False),
    )(ht, src, dst)


def _dinv_from_hist(hist_blk):
    return lax.rsqrt(jnp.sum(hist_blk, axis=0) + 1.0)


def _tc1_body(hist_ref, x_ref, w1_ref, out_ref):
    dinv = _dinv_from_hist(hist_ref[...])
    out_ref[...] = jnp.dot(x_ref[...] * dinv[:, None], w1_ref[...],
                           preferred_element_type=jnp.float32)


def _tc1(hist, x, W1):
    return pl.pallas_call(
        _tc1_body,
        out_shape=jax.ShapeDtypeStruct((N_NODES, 128), jnp.float32),
    )(hist, x, W1)


def _tc2_body(hist_ref, p_ref, h_ref, b1_ref, a_ref, w2t_ref, out_ref):
    dinv = _dinv_from_hist(hist_ref[...])
    s = (p_ref[0] + p_ref[1] - h_ref[...]) * dinv[:, None] + b1_ref[...]
    f = jnp.where(s >= 0, s, a_ref[0, 0] * s) * dinv[:, None]
    out_ref[...] = lax.dot_general(
        w2t_ref[...], f, (((1,), (1,)), ((), ())),
        preferred_element_type=jnp.float32)


def _tc2(hist, p, h1p, b1, a, W2t):
    h2 = W2t.shape[0]
    return pl.pallas_call(
        _tc2_body,
        out_shape=jax.ShapeDtypeStruct((h2, N_NODES), jnp.float32),
    )(hist, p, h1p, b1.reshape(1, 128), a.reshape(1, 1), W2t)


def _tc3_body(hist_ref, q_ref, ht_ref, b2_ref, a_ref, fcw_ref, fcb_ref,
              out_ref):
    dinv = _dinv_from_hist(hist_ref[...])
    s = (q_ref[0] + q_ref[1] + ht_ref[...]) * dinv[None, :] + b2_ref[...]
    f = jnp.where(s >= 0, s, a_ref[0, 0] * s)
    out_ref[...] = lax.dot_general(
        f, fcw_ref[...], (((0,), (0,)), ((), ())),
        preferred_element_type=jnp.float32) + fcb_ref[...]


def _tc3(hist, q, h2pt, b2, a, fcW, fcb):
    h2, out_dim = fcW.shape
    return pl.pallas_call(
        _tc3_body,
        out_shape=jax.ShapeDtypeStruct((N_NODES, out_dim), jnp.float32),
    )(hist, q, h2pt, b2.reshape(h2, 1), a.reshape(1, 1), fcW,
      fcb.reshape(1, out_dim))


@jax.jit
def kernel(x, edge_index, W1, b1, W2, b2, a, fcW, fcb):
    src_flat = edge_index[0]
    dst_flat = edge_index[1]
    src = src_flat.reshape(NW * ROWS_PER_W, CHUNK)
    dst = dst_flat.reshape(NW * ROWS_PER_W, CHUNK)

    hist = _sc_degree(dst_flat)

    h1p = _tc1(hist, x, W1)
    p = _sc_aggregate(h1p, src, dst)
    h2pt = _tc2(hist, p, h1p, b1, a, W2.T)
    q = _sc_aggregate16(h2pt, src_flat, dst_flat)
    return _tc3(hist, q, h2pt, b2, a, fcW, fcb)


# speedup vs baseline: 34.7662x; 1.5632x over previous
import jax
import jax.numpy as jnp
from jax import lax
from jax.experimental import pallas as pl
from jax.experimental.pallas import tpu as pltpu
from jax.experimental.pallas import tpu_sc as plsc

N_NODES = 10000
N_EDGES = 320000

NC = 2
NS = 16
NW = NC * NS

E_PER_W = N_EDGES // NW
CHUNK = 125
ROWS_PER_W = E_PER_W // CHUNK
HALF_ROWS = ROWS_PER_W // 2
SLAB = 624
TAIL_BASE = SLAB * NS
TAIL = N_NODES - TAIL_BASE


def _sc_mesh():
    return plsc.VectorSubcoreMesh(core_axis_name="c", subcore_axis_name="s")


def _deg_body(dst_hbm, hist_hbm, dst_v, hist_v):
    wid = lax.axis_index("s") * NC + lax.axis_index("c")
    ones = jnp.ones((16,), jnp.float32)
    zeros = jnp.zeros((16,), jnp.float32)

    @pl.loop(0, N_NODES // 16)
    def _(i):
        hist_v[pl.ds(i * 16, 16)] = zeros

    pltpu.sync_copy(dst_hbm.at[pl.ds(wid * E_PER_W, E_PER_W)], dst_v)

    @pl.loop(0, E_PER_W // 16)
    def _(i):
        idx = dst_v[pl.ds(i * 16, 16)]
        plsc.addupdate_scatter(hist_v, [idx], ones)

    pltpu.sync_copy(hist_v, hist_hbm.at[wid])


def _sc_degree(dst):
    return pl.kernel(
        _deg_body,
        out_type=jax.ShapeDtypeStruct((NW, N_NODES), jnp.float32),
        mesh=_sc_mesh(),
        scratch_types=[
            pltpu.VMEM((E_PER_W,), jnp.int32),
            pltpu.VMEM((N_NODES,), jnp.float32),
        ],
        compiler_params=pltpu.CompilerParams(needs_layout_passes=False),
    )(dst)


def _agg_body(h_hbm, src_hbm, dst_hbm, out_hbm, acc, src_v, dst_v, rbuf, sem):
    cid = lax.axis_index("c")
    sid = lax.axis_index("s")
    wid = sid * NC + cid

    nbase = sid * SLAB
    pltpu.sync_copy(h_hbm.at[pl.ds(nbase, SLAB)], acc.at[pl.ds(nbase, SLAB)])

    @pl.when(sid == NS - 1)
    def _():
        pltpu.sync_copy(h_hbm.at[pl.ds(TAIL_BASE, TAIL)],
                        acc.at[pl.ds(TAIL_BASE, TAIL)])

    plsc.subcore_barrier()

    for half in range(2):
        rbase = wid * ROWS_PER_W + half * HALF_ROWS
        pltpu.sync_copy(src_hbm.at[pl.ds(rbase, HALF_ROWS)], src_v)
        pltpu.sync_copy(dst_hbm.at[pl.ds(rbase, HALF_ROWS)], dst_v)

        pltpu.async_copy(h_hbm.at[src_v.at[0]], rbuf[0], sem[0])

        @pl.loop(0, HALF_ROWS // 2)
        def _(jj):
            j0 = jj * 2
            j1 = j0 + 1
            pltpu.make_async_copy(
                h_hbm.at[src_v.at[j0]], rbuf[0], sem[0]).wait()
            pltpu.async_copy(h_hbm.at[src_v.at[j1]], rbuf[1], sem[1])
            pltpu.sync_copy(rbuf[0], acc.at[dst_v.at[j0]], add=True)
            pltpu.make_async_copy(
                h_hbm.at[src_v.at[j1]], rbuf[1], sem[1]).wait()

            @pl.when(jj < HALF_ROWS // 2 - 1)
            def _():
                pltpu.async_copy(h_hbm.at[src_v.at[j0 + 2]], rbuf[0], sem[0])

            pltpu.sync_copy(rbuf[1], acc.at[dst_v.at[j1]], add=True)

    plsc.subcore_barrier()

    pltpu.sync_copy(acc.at[pl.ds(nbase, SLAB)],
                    out_hbm.at[cid, pl.ds(nbase, SLAB)])

    @pl.when(sid == NS - 1)
    def _():
        pltpu.sync_copy(acc.at[pl.ds(TAIL_BASE, TAIL)],
                        out_hbm.at[cid, pl.ds(TAIL_BASE, TAIL)])


def _sc_aggregate(h, src2d, dst2d):
    d = h.shape[1]
    return pl.kernel(
        _agg_body,
        out_type=jax.ShapeDtypeStruct((NC, N_NODES, d), jnp.float32),
        mesh=_sc_mesh(),
        scratch_types=[
            pltpu.VMEM_SHARED((N_NODES, d), jnp.float32),
            pltpu.VMEM((HALF_ROWS, CHUNK), jnp.int32),
            pltpu.VMEM((HALF_ROWS, CHUNK), jnp.int32),
            [pltpu.VMEM((CHUNK, d), jnp.float32) for _ in range(2)],
            [pltpu.SemaphoreType.DMA for _ in range(2)],
        ],
    )(h, src2d, dst2d)


_ECHUNK = 2000
_CPW = 4
_NES = NW * _CPW // 16
E_SLICE = N_EDGES // _NES


def _agg16_body(ht_hbm, src_hbm, dst_hbm, out_hbm, tbl, acc, src_v, dst_v):
    cid = lax.axis_index("c")
    sid = lax.axis_index("s")
    wid = sid * NC + cid
    cgrp = wid % (16 // _CPW)
    esl = wid // (16 // _CPW)

    zeros = jnp.zeros((16,), jnp.float32)

    for k in range(_CPW):
        pltpu.sync_copy(ht_hbm.at[cgrp * _CPW + k], tbl[k])

        @pl.loop(0, N_NODES // 16)
        def _(i):
            acc[k][pl.ds(i * 16, 16)] = zeros

    @pl.loop(0, E_SLICE // _ECHUNK)
    def _(kk):
        ebase = esl * E_SLICE + kk * _ECHUNK
        pltpu.sync_copy(src_hbm.at[pl.ds(ebase, _ECHUNK)], src_v)
        pltpu.sync_copy(dst_hbm.at[pl.ds(ebase, _ECHUNK)], dst_v)

        @pl.loop(0, _ECHUNK // 16, unroll=4)
        def _(i):
            idx_s = src_v[pl.ds(i * 16, 16)]
            idx_d = dst_v[pl.ds(i * 16, 16)]
            for k in range(_CPW):
                vals = plsc.load_gather(tbl[k], [idx_s])
                plsc.addupdate_scatter(acc[k], [idx_d], vals)

    for k in range(_CPW):
        pltpu.sync_copy(acc[k], out_hbm.at[esl, cgrp * _CPW + k])


def _sc_aggregate16(ht, src, dst):
    h2 = ht.shape[0]
    return pl.kernel(
        _agg16_body,
        out_type=jax.ShapeDtypeStruct((_NES, h2, N_NODES), jnp.float32),
        mesh=_sc_mesh(),
        scratch_types=[
            [pltpu.VMEM((N_NODES,), jnp.float32) for _ in range(_CPW)],
            [pltpu.VMEM((N_NODES,), jnp.float32) for _ in range(_CPW)],
            pltpu.VMEM((_ECHUNK,), jnp.int32),
            pltpu.VMEM((_ECHUNK,), jnp.int32),
        ],
        compiler_params=pltpu.CompilerParams(needs_layout_passes=False),
    )(ht, src, dst)


def _dinv_from_hist(hist_blk):
    return lax.rsqrt(jnp.sum(hist_blk, axis=0) + 1.0)


def _tc1_body(hist_ref, x_ref, w1_ref, out_ref):
    dinv = _dinv_from_hist(hist_ref[...])
    out_ref[...] = jnp.dot(x_ref[...] * dinv[:, None], w1_ref[...],
                           preferred_element_type=jnp.float32)


def _tc1(hist, x, W1):
    return pl.pallas_call(
        _tc1_body,
        out_shape=jax.ShapeDtypeStruct((N_NODES, 128), jnp.float32),
    )(hist, x, W1)


def _tc2_body(hist_ref, p_ref, h_ref, b1_ref, a_ref, w2t_ref, out_ref):
    dinv = _dinv_from_hist(hist_ref[...])
    s = (p_ref[0] + p_ref[1] - h_ref[...]) * dinv[:, None] + b1_ref[...]
    f = jnp.where(s >= 0, s, a_ref[0, 0] * s) * dinv[:, None]
    out_ref[...] = lax.dot_general(
        w2t_ref[...], f, (((1,), (1,)), ((), ())),
        preferred_element_type=jnp.float32)


def _tc2(hist, p, h1p, b1, a, W2t):
    h2 = W2t.shape[0]
    return pl.pallas_call(
        _tc2_body,
        out_shape=jax.ShapeDtypeStruct((h2, N_NODES), jnp.float32),
    )(hist, p, h1p, b1.reshape(1, 128), a.reshape(1, 1), W2t)


def _tc3_body(hist_ref, q_ref, ht_ref, b2_ref, a_ref, fcw_ref, fcb_ref,
              out_ref):
    dinv = _dinv_from_hist(hist_ref[...])
    s = (jnp.sum(q_ref[...], axis=0) + ht_ref[...]) * dinv[None, :] + b2_ref[...]
    f = jnp.where(s >= 0, s, a_ref[0, 0] * s)
    out_ref[...] = lax.dot_general(
        f, fcw_ref[...], (((0,), (0,)), ((), ())),
        preferred_element_type=jnp.float32) + fcb_ref[...]


def _tc3(hist, q, h2pt, b2, a, fcW, fcb):
    h2, out_dim = fcW.shape
    return pl.pallas_call(
        _tc3_body,
        out_shape=jax.ShapeDtypeStruct((N_NODES, out_dim), jnp.float32),
    )(hist, q, h2pt, b2.reshape(h2, 1), a.reshape(1, 1), fcW,
      fcb.reshape(1, out_dim))


@jax.jit
def kernel(x, edge_index, W1, b1, W2, b2, a, fcW, fcb):
    src_flat = edge_index[0]
    dst_flat = edge_index[1]
    src = src_flat.reshape(NW * ROWS_PER_W, CHUNK)
    dst = dst_flat.reshape(NW * ROWS_PER_W, CHUNK)

    hist = _sc_degree(dst_flat)

    h1p = _tc1(hist, x, W1)
    p = _sc_aggregate(h1p, src, dst)
    h2pt = _tc2(hist, p, h1p, b1, a, W2.T)
    q = _sc_aggregate16(h2pt, src_flat, dst_flat)
    return _tc3(hist, q, h2pt, b2, a, fcW, fcb)


# speedup vs baseline: 37.9320x; 1.0911x over previous
import jax
import jax.numpy as jnp
from jax import lax
from jax.experimental import pallas as pl
from jax.experimental.pallas import tpu as pltpu
from jax.experimental.pallas import tpu_sc as plsc

N_NODES = 10000
N_EDGES = 320000

NC = 2
NS = 16
NW = NC * NS

E_PER_W = N_EDGES // NW
CHUNK = 125
ROWS_PER_W = E_PER_W // CHUNK
HALF_ROWS = ROWS_PER_W // 2
SLAB = 624
TAIL_BASE = SLAB * NS
TAIL = N_NODES - TAIL_BASE


def _sc_mesh():
    return plsc.VectorSubcoreMesh(core_axis_name="c", subcore_axis_name="s")


def _deg_body(dst_hbm, hist_hbm, dst_v, hist_v, sem):
    wid = lax.axis_index("s") * NC + lax.axis_index("c")
    ones = jnp.ones((16,), jnp.float32)
    zeros = jnp.zeros((16,), jnp.float32)

    cp = pltpu.async_copy(dst_hbm.at[pl.ds(wid * E_PER_W, E_PER_W)], dst_v,
                          sem)

    @pl.loop(0, N_NODES // 16, unroll=8)
    def _(i):
        hist_v[pl.ds(i * 16, 16)] = zeros

    cp.wait()

    @pl.loop(0, E_PER_W // 16, unroll=8)
    def _(i):
        idx = dst_v[pl.ds(i * 16, 16)]
        plsc.addupdate_scatter(hist_v, [idx], ones)

    pltpu.sync_copy(hist_v, hist_hbm.at[wid])


def _sc_degree(dst):
    return pl.kernel(
        _deg_body,
        out_type=jax.ShapeDtypeStruct((NW, N_NODES), jnp.float32),
        mesh=_sc_mesh(),
        scratch_types=[
            pltpu.VMEM((E_PER_W,), jnp.int32),
            pltpu.VMEM((N_NODES,), jnp.float32),
            pltpu.SemaphoreType.DMA,
        ],
        compiler_params=pltpu.CompilerParams(needs_layout_passes=False),
    )(dst)


def _agg_body(h_hbm, src_hbm, dst_hbm, out_hbm, acc, src_v, dst_v, rbuf, sem,
              sem_i):
    cid = lax.axis_index("c")
    sid = lax.axis_index("s")
    wid = sid * NC + cid

    nbase = sid * SLAB
    ci = pltpu.async_copy(h_hbm.at[pl.ds(nbase, SLAB)],
                          acc.at[pl.ds(nbase, SLAB)], sem_i[2])

    @pl.when(sid == NS - 1)
    def _():
        pltpu.sync_copy(h_hbm.at[pl.ds(TAIL_BASE, TAIL)],
                        acc.at[pl.ds(TAIL_BASE, TAIL)])

    for half in range(2):
        rbase = wid * ROWS_PER_W + half * HALF_ROWS
        cs = pltpu.async_copy(src_hbm.at[pl.ds(rbase, HALF_ROWS)], src_v,
                              sem_i[0])
        cd = pltpu.async_copy(dst_hbm.at[pl.ds(rbase, HALF_ROWS)], dst_v,
                              sem_i[1])
        if half == 0:
            ci.wait()
            plsc.subcore_barrier()
        cs.wait()
        cd.wait()

        pltpu.async_copy(h_hbm.at[src_v.at[0]], rbuf[0], sem[0])

        @pl.loop(0, HALF_ROWS // 2)
        def _(jj):
            j0 = jj * 2
            j1 = j0 + 1
            pltpu.make_async_copy(
                h_hbm.at[src_v.at[j0]], rbuf[0], sem[0]).wait()
            pltpu.async_copy(h_hbm.at[src_v.at[j1]], rbuf[1], sem[1])
            pltpu.sync_copy(rbuf[0], acc.at[dst_v.at[j0]], add=True)
            pltpu.make_async_copy(
                h_hbm.at[src_v.at[j1]], rbuf[1], sem[1]).wait()

            @pl.when(jj < HALF_ROWS // 2 - 1)
            def _():
                pltpu.async_copy(h_hbm.at[src_v.at[j0 + 2]], rbuf[0], sem[0])

            pltpu.sync_copy(rbuf[1], acc.at[dst_v.at[j1]], add=True)

    plsc.subcore_barrier()

    pltpu.sync_copy(acc.at[pl.ds(nbase, SLAB)],
                    out_hbm.at[cid, pl.ds(nbase, SLAB)])

    @pl.when(sid == NS - 1)
    def _():
        pltpu.sync_copy(acc.at[pl.ds(TAIL_BASE, TAIL)],
                        out_hbm.at[cid, pl.ds(TAIL_BASE, TAIL)])


def _sc_aggregate(h, src2d, dst2d):
    d = h.shape[1]
    return pl.kernel(
        _agg_body,
        out_type=jax.ShapeDtypeStruct((NC, N_NODES, d), jnp.float32),
        mesh=_sc_mesh(),
        scratch_types=[
            pltpu.VMEM_SHARED((N_NODES, d), jnp.float32),
            pltpu.VMEM((HALF_ROWS, CHUNK), jnp.int32),
            pltpu.VMEM((HALF_ROWS, CHUNK), jnp.int32),
            [pltpu.VMEM((CHUNK, d), jnp.float32) for _ in range(2)],
            [pltpu.SemaphoreType.DMA for _ in range(2)],
            [pltpu.SemaphoreType.DMA for _ in range(3)],
        ],
    )(h, src2d, dst2d)


_ECHUNK = 4000
_CPW = 4
_NES = NW * _CPW // 16
E_SLICE = N_EDGES // _NES
_NCH = E_SLICE // _ECHUNK


def _agg16_body(ht_hbm, src_hbm, dst_hbm, out_hbm, tbl, acc, src_v, dst_v,
                sem_s, sem_d, sem_t):
    cid = lax.axis_index("c")
    sid = lax.axis_index("s")
    wid = sid * NC + cid
    cgrp = wid % (16 // _CPW)
    esl = wid // (16 // _CPW)

    zeros = jnp.zeros((16,), jnp.float32)
    ebase = esl * E_SLICE

    ct = [pltpu.async_copy(ht_hbm.at[cgrp * _CPW + k], tbl[k], sem_t[k])
          for k in range(_CPW)]
    pltpu.async_copy(src_hbm.at[pl.ds(ebase, _ECHUNK)], src_v[0], sem_s[0])
    pltpu.async_copy(dst_hbm.at[pl.ds(ebase, _ECHUNK)], dst_v[0], sem_d[0])

    for k in range(_CPW):

        @pl.loop(0, N_NODES // 16, unroll=8)
        def _(i):
            acc[k][pl.ds(i * 16, 16)] = zeros

        ct[k].wait()

    def _process(kk, b):
        pltpu.make_async_copy(
            src_hbm.at[pl.ds(ebase + kk * _ECHUNK, _ECHUNK)], src_v[b],
            sem_s[b]).wait()
        pltpu.make_async_copy(
            dst_hbm.at[pl.ds(ebase + kk * _ECHUNK, _ECHUNK)], dst_v[b],
            sem_d[b]).wait()

        @pl.when(kk < _NCH - 1)
        def _():
            nb = ebase + (kk + 1) * _ECHUNK
            pltpu.async_copy(src_hbm.at[pl.ds(nb, _ECHUNK)], src_v[1 - b],
                             sem_s[1 - b])
            pltpu.async_copy(dst_hbm.at[pl.ds(nb, _ECHUNK)], dst_v[1 - b],
                             sem_d[1 - b])

        @pl.loop(0, _ECHUNK // 16, unroll=8)
        def _(i):
            idx_s = src_v[b][pl.ds(i * 16, 16)]
            idx_d = dst_v[b][pl.ds(i * 16, 16)]
            for k in range(_CPW):
                vals = plsc.load_gather(tbl[k], [idx_s])
                plsc.addupdate_scatter(acc[k], [idx_d], vals)

    @pl.loop(0, _NCH // 2)
    def _(jj):
        _process(jj * 2, 0)
        _process(jj * 2 + 1, 1)

    for k in range(_CPW):
        pltpu.sync_copy(acc[k], out_hbm.at[esl, cgrp * _CPW + k])


def _sc_aggregate16(ht, src, dst):
    h2 = ht.shape[0]
    return pl.kernel(
        _agg16_body,
        out_type=jax.ShapeDtypeStruct((_NES, h2, N_NODES), jnp.float32),
        mesh=_sc_mesh(),
        scratch_types=[
            [pltpu.VMEM((N_NODES,), jnp.float32) for _ in range(_CPW)],
            [pltpu.VMEM((N_NODES,), jnp.float32) for _ in range(_CPW)],
            [pltpu.VMEM((_ECHUNK,), jnp.int32) for _ in range(2)],
            [pltpu.VMEM((_ECHUNK,), jnp.int32) for _ in range(2)],
            [pltpu.SemaphoreType.DMA for _ in range(2)],
            [pltpu.SemaphoreType.DMA for _ in range(2)],
            [pltpu.SemaphoreType.DMA for _ in range(_CPW)],
        ],
        compiler_params=pltpu.CompilerParams(needs_layout_passes=False),
    )(ht, src, dst)


def _dinv_from_hist(hist_blk):
    return lax.rsqrt(jnp.sum(hist_blk, axis=0) + 1.0)


def _tc1_body(hist_ref, x_ref, w1_ref, out_ref):
    dinv = _dinv_from_hist(hist_ref[...])
    out_ref[...] = jnp.dot(x_ref[...] * dinv[:, None], w1_ref[...],
                           preferred_element_type=jnp.float32)


def _tc1(hist, x, W1):
    return pl.pallas_call(
        _tc1_body,
        out_shape=jax.ShapeDtypeStruct((N_NODES, 128), jnp.float32),
    )(hist, x, W1)


def _tc2_body(hist_ref, p_ref, h_ref, b1_ref, a_ref, w2t_ref, out_ref):
    dinv = _dinv_from_hist(hist_ref[...])
    s = (p_ref[0] + p_ref[1] - h_ref[...]) * dinv[:, None] + b1_ref[...]
    f = jnp.where(s >= 0, s, a_ref[0, 0] * s) * dinv[:, None]
    out_ref[...] = lax.dot_general(
        w2t_ref[...], f, (((1,), (1,)), ((), ())),
        preferred_element_type=jnp.float32)


def _tc2(hist, p, h1p, b1, a, W2t):
    h2 = W2t.shape[0]
    return pl.pallas_call(
        _tc2_body,
        out_shape=jax.ShapeDtypeStruct((h2, N_NODES), jnp.float32),
    )(hist, p, h1p, b1.reshape(1, 128), a.reshape(1, 1), W2t)


def _tc3_body(hist_ref, q_ref, ht_ref, b2_ref, a_ref, fcw_ref, fcb_ref,
              out_ref):
    dinv = _dinv_from_hist(hist_ref[...])
    s = (jnp.sum(q_ref[...], axis=0) + ht_ref[...]) * dinv[None, :] + b2_ref[...]
    f = jnp.where(s >= 0, s, a_ref[0, 0] * s)
    out_ref[...] = lax.dot_general(
        f, fcw_ref[...], (((0,), (0,)), ((), ())),
        preferred_element_type=jnp.float32) + fcb_ref[...]


def _tc3(hist, q, h2pt, b2, a, fcW, fcb):
    h2, out_dim = fcW.shape
    return pl.pallas_call(
        _tc3_body,
        out_shape=jax.ShapeDtypeStruct((N_NODES, out_dim), jnp.float32),
    )(hist, q, h2pt, b2.reshape(h2, 1), a.reshape(1, 1), fcW,
      fcb.reshape(1, out_dim))


@jax.jit
def kernel(x, edge_index, W1, b1, W2, b2, a, fcW, fcb):
    src_flat = edge_index[0]
    dst_flat = edge_index[1]
    src = src_flat.reshape(NW * ROWS_PER_W, CHUNK)
    dst = dst_flat.reshape(NW * ROWS_PER_W, CHUNK)

    hist = _sc_degree(dst_flat)

    h1p = _tc1(hist, x, W1)
    p = _sc_aggregate(h1p, src, dst)
    h2pt = _tc2(hist, p, h1p, b1, a, W2.T)
    q = _sc_aggregate16(h2pt, src_flat, dst_flat)
    return _tc3(hist, q, h2pt, b2, a, fcW, fcb)
